# baseline (speedup 1.0000x reference)
import jax
import jax.numpy as jnp
from jax import lax
from jax.experimental import pallas as pl
from jax.experimental.pallas import tpu as pltpu
from jax.experimental.pallas import tpu_sc as plsc

D_MODEL = 768
MAX_REL = 128
VOCAB = 2 * MAX_REL + 1
S = 512

NC = 2
NS = 16
NW = NC * NS
ROWS_PER_W = S // NW
CHUNK = 64
NCHUNK = S // CHUNK

STAGE = 79
PAD = 64
BUF_ROWS = PAD + STAGE
NVEC = D_MODEL // 16


def _replicate_row(buf, src_row, dst_base):
    vals = [buf[src_row, pl.ds(g * 16, 16)] for g in range(NVEC)]

    def body(r, carry):
        for g in range(NVEC):
            buf[dst_base + r, pl.ds(g * 16, 16)] = vals[g]
        return carry

    lax.fori_loop(0, PAD, body, 0)


def _rpe_body(table_hbm, out_hbm, buf):
    wid = lax.axis_index("s") * NC + lax.axis_index("c")
    i0 = wid * ROWS_PER_W

    def chunk_step(c, has_staged_r):
        j0 = c * CHUNK
        rel = j0 - i0
        is_left = rel < -(MAX_REL - ROWS_PER_W + 1)
        is_right = rel > MAX_REL - CHUNK + 1
        is_mid = jnp.logical_not(jnp.logical_or(is_left, is_right))

        @pl.when(jnp.logical_and(is_left, c == 0))
        def _():
            pltpu.sync_copy(table_hbm.at[pl.ds(0, STAGE)],
                            buf.at[pl.ds(PAD, STAGE)])
            _replicate_row(buf, PAD, 0)

        @pl.when(jnp.logical_and(is_right, has_staged_r == 0))
        def _():
            pltpu.sync_copy(table_hbm.at[pl.ds(VOCAB - STAGE, STAGE)],
                            buf.at[pl.ds(0, STAGE)])
            _replicate_row(buf, STAGE - 1, STAGE)

        @pl.when(is_mid)
        def _():
            te = rel + MAX_REL - (ROWS_PER_W - 1)
            pltpu.sync_copy(table_hbm.at[pl.ds(te, STAGE)],
                            buf.at[pl.ds(PAD, STAGE)])

        for r in range(ROWS_PER_W):
            i = i0 + r
            lo = j0 - i + MAX_REL
            x_left = jnp.maximum(lo + PAD, 0)
            x_right = jnp.minimum(lo - (VOCAB - STAGE), STAGE)
            x_mid = STAGE - r
            x = jnp.where(is_left, x_left, jnp.where(is_right, x_right, x_mid))
            pltpu.sync_copy(buf.at[pl.ds(x, CHUNK)],
                            out_hbm.at[pl.ds(i * S + j0, CHUNK)])

        return jnp.where(is_right, 1, has_staged_r)

    lax.fori_loop(0, NCHUNK, chunk_step, 0)


def kernel(seq_len, table):
    out = pl.kernel(
        _rpe_body,
        mesh=plsc.VectorSubcoreMesh(core_axis_name="c", subcore_axis_name="s"),
        out_type=jax.ShapeDtypeStruct((S * S, D_MODEL), jnp.float32),
        scratch_types=[
            pltpu.VMEM((BUF_ROWS, D_MODEL), jnp.float32),
        ],
        compiler_params=pltpu.CompilerParams(use_tc_tiling_on_sc=False),
    )(table)
    return out.reshape(S, S, D_MODEL)
